# baseline (speedup 1.0000x reference)
import jax
import jax.numpy as jnp
from jax.experimental import pallas as pl
from jax.experimental.pallas import tpu as pltpu

_N = 10000
_BLK = 1000


def _col_sum_kernel(x_ref, o_ref, acc_ref):
    i = pl.program_id(0)

    @pl.when(i == 0)
    def _():
        acc_ref[0] = 0.0

    acc_ref[0] += jnp.sum(x_ref[:, 1])

    @pl.when(i == pl.num_programs(0) - 1)
    def _():
        o_ref[0, 0] = jnp.sqrt(acc_ref[0])


def kernel(node_attr, edgeij_pair, edge_attr, g, batch):
    out = pl.pallas_call(
        _col_sum_kernel,
        grid=(_N // _BLK,),
        in_specs=[pl.BlockSpec((_BLK, 128), lambda i: (i, 0))],
        out_specs=pl.BlockSpec((1, 1), lambda i: (0, 0), memory_space=pltpu.SMEM),
        out_shape=jax.ShapeDtypeStruct((1, 1), jnp.float32),
        scratch_shapes=[pltpu.SMEM((1,), jnp.float32)],
    )(node_attr)
    return out[0, 0]
